# baseline (speedup 1.0000x reference)
import functools

import jax
import jax.numpy as jnp
from jax import lax
from jax.experimental import pallas as pl
from jax.experimental.pallas import tpu as pltpu
from jax.experimental.pallas import tpu_sc as plsc

_NUM_NODES = 1000000
_DIM = 64
_BATCH = 16384
_NC, _NS, _L = 2, 16, 16
_NW = _NC * _NS
_BPW = _BATCH // _NW
_CHUNK = 128
_NCHUNK = _BPW // _CHUNK
_HALF = _BPW // 2
_NLINES = _NUM_NODES // 2

_LBASE = 15624
_WLINES = 128
_WROWS = 2 * _WLINES
_NWIN = 123

_LN2 = 0.6931471805599453
_SQRT2 = 1.4142135623730951


def _neg_log(p):
    bits = plsc.bitcast(p, jnp.int32)
    e = lax.shift_right_logical(bits, 23) - 127
    m = plsc.bitcast(
        (bits & jnp.int32(0x007FFFFF)) | jnp.int32(0x3F800000), jnp.float32
    )
    big = m > jnp.float32(_SQRT2)
    m = jnp.where(big, m * jnp.float32(0.5), m)
    e = jnp.where(big, e + 1, e)
    s = (m - jnp.float32(1.0)) / (m + jnp.float32(1.0))
    s2 = s * s
    lnm = s * (jnp.float32(2.0) + s2 * (jnp.float32(2.0 / 3.0)
               + s2 * (jnp.float32(2.0 / 5.0) + s2 * jnp.float32(2.0 / 7.0))))
    return -(e.astype(jnp.float32) * jnp.float32(_LN2) + lnm)


def _pack_body(table_hbm, lines_hbm, in_buf, out_buf, rsems, wsems):
    wid = lax.axis_index("s") * _NC + lax.axis_index("c")
    line0 = wid * _LBASE + jnp.minimum(wid, 4) * 8
    nlines = _LBASE + jnp.where(wid < 4, 8, 0)

    def win_start(w):
        return line0 + jnp.minimum(w * _WLINES, nlines - _WLINES)

    def start_read(w, par):
        src = table_hbm.at[pl.ds(2 * win_start(w), _WROWS)]
        pltpu.async_copy(src, in_buf.at[pl.ds(par * _WROWS, _WROWS)],
                         rsems[par])

    def drain(sem, dst_slice, src_slice):
        pltpu.make_async_copy(src_slice, dst_slice, sem).wait()

    start_read(0, 0)

    def do_window(w, par):
        npar = 1 - par
        @pl.when(w < _NWIN - 1)
        def _():
            start_read(w + 1, npar)
        drain(rsems[par], in_buf.at[pl.ds(par * _WROWS, _WROWS)],
              table_hbm.at[pl.ds(0, _WROWS)])
        @pl.when(w >= 2)
        def _():
            drain(wsems[par], out_buf.at[pl.ds(par * _WLINES, _WLINES)],
                  lines_hbm.at[pl.ds(0, _WLINES)])
        poff = par * _WROWS
        ooff = par * _WLINES
        for r in range(_WROWS):
            orow = ooff + r // 2
            ocol = (r % 2) * _DIM
            for k in range(_DIM // _L):
                out_buf[orow, pl.ds(ocol + k * _L, _L)] = (
                    in_buf[poff + r, pl.ds(k * _L, _L)])
        pltpu.async_copy(
            out_buf.at[pl.ds(ooff, _WLINES)],
            lines_hbm.at[pl.ds(win_start(w), _WLINES)],
            wsems[par])

    def dbl(t, carry):
        do_window(2 * t, 0)
        do_window(2 * t + 1, 1)
        return carry

    lax.fori_loop(0, (_NWIN - 1) // 2, dbl, None)
    do_window(_NWIN - 1, (_NWIN - 1) % 2)
    for par in range(2):
        drain(wsems[par], out_buf.at[pl.ds(par * _WLINES, _WLINES)],
              lines_hbm.at[pl.ds(0, _WLINES)])


@functools.partial(
    pl.kernel,
    out_type=jax.ShapeDtypeStruct((_NLINES, 128), jnp.float32),
    mesh=plsc.VectorSubcoreMesh(
        core_axis_name="c", subcore_axis_name="s",
        num_cores=_NC, num_subcores=_NS),
    scratch_types=[
        pltpu.VMEM((2 * _WROWS, _DIM), jnp.float32),
        pltpu.VMEM((2 * _WLINES, 128), jnp.float32),
        [pltpu.SemaphoreType.DMA, pltpu.SemaphoreType.DMA],
        [pltpu.SemaphoreType.DMA, pltpu.SemaphoreType.DMA],
    ],
    compiler_params=pltpu.CompilerParams(needs_layout_passes=False),
)
def _sc_pack(table_hbm, lines_hbm, *scratch):
    _pack_body(table_hbm, lines_hbm, *scratch)


def _loss_body(lines_hbm, hi_i_hbm, hi_j_hbm, idxi_hbm, idxj_hbm, out_hbm,
               hi_i_v, hi_j_v, idxi_v, idxj_v, rows_i, rows_j, out_v, sem):
    wid = lax.axis_index("s") * _NC + lax.axis_index("c")

    pltpu.sync_copy(hi_i_hbm.at[wid], hi_i_v)
    pltpu.sync_copy(hi_j_hbm.at[wid], hi_j_v)
    pltpu.sync_copy(idxi_hbm.at[wid], idxi_v)
    pltpu.sync_copy(idxj_hbm.at[wid], idxj_v)

    lanes = lax.iota(jnp.int32, _L)

    def run_half(h):
        copies = []
        for c in range(_NCHUNK // 2):
            chunk = h * (_NCHUNK // 2) + c
            dst = pl.ds(c * _CHUNK, _CHUNK)
            copies.append(pltpu.async_copy(
                lines_hbm.at[hi_i_v.at[chunk]], rows_i.at[dst], sem))
            copies.append(pltpu.async_copy(
                lines_hbm.at[hi_j_v.at[chunk]], rows_j.at[dst], sem))
        for cp in copies:
            cp.wait()

        for g in range(_HALF // _L):
            base = h * _HALF + g * _L
            row_idx = lanes + g * _L
            pi = (idxi_v[pl.ds(base, _L)] & 1) * _DIM
            pj = (idxj_v[pl.ds(base, _L)] & 1) * _DIM
            acc = jnp.zeros((_L,), jnp.float32)
            for d in range(_DIM):
                vi = plsc.load_gather(rows_i, [row_idx, pi + d])
                vj = plsc.load_gather(rows_j, [row_idx, pj + d])
                acc = acc + vi * vj
            p = jnp.float32(1.0) / (jnp.float32(1.0) + jnp.exp(-acc))
            p = jnp.maximum(p, jnp.float32(1e-8))
            out_v[pl.ds(base, _L)] = _neg_log(p)

    run_half(0)
    run_half(1)
    pltpu.sync_copy(out_v, out_hbm.at[pl.ds(wid * _BPW, _BPW)])


@functools.partial(
    pl.kernel,
    out_type=jax.ShapeDtypeStruct((_BATCH,), jnp.float32),
    mesh=plsc.VectorSubcoreMesh(
        core_axis_name="c", subcore_axis_name="s",
        num_cores=_NC, num_subcores=_NS),
    scratch_types=[
        pltpu.VMEM((_NCHUNK, _CHUNK), jnp.int32),
        pltpu.VMEM((_NCHUNK, _CHUNK), jnp.int32),
        pltpu.VMEM((_BPW,), jnp.int32),
        pltpu.VMEM((_BPW,), jnp.int32),
        pltpu.VMEM((_HALF, 128), jnp.float32),
        pltpu.VMEM((_HALF, 128), jnp.float32),
        pltpu.VMEM((_BPW,), jnp.float32),
        pltpu.SemaphoreType.DMA,
    ],
    compiler_params=pltpu.CompilerParams(needs_layout_passes=False),
)
def _sc_loss(lines_hbm, hi_i_hbm, hi_j_hbm, idxi_hbm, idxj_hbm, out_hbm,
             *scratch):
    _loss_body(lines_hbm, hi_i_hbm, hi_j_hbm, idxi_hbm, idxj_hbm, out_hbm,
               *scratch)


def kernel(node_i, node_j, table):
    idx_i = node_i.astype(jnp.int32)
    idx_j = node_j.astype(jnp.int32)
    hi_i = (idx_i >> 1).reshape(_NW, _NCHUNK, _CHUNK)
    hi_j = (idx_j >> 1).reshape(_NW, _NCHUNK, _CHUNK)
    lines = _sc_pack(table)
    return _sc_loss(lines, hi_i, hi_j,
                    idx_i.reshape(_NW, _BPW), idx_j.reshape(_NW, _BPW))


# speedup vs baseline: 1.7716x; 1.7716x over previous
import functools

import jax
import jax.numpy as jnp
from jax import lax
from jax.experimental import pallas as pl
from jax.experimental.pallas import tpu as pltpu
from jax.experimental.pallas import tpu_sc as plsc

_NUM_NODES = 1000000
_DIM = 64
_BATCH = 16384
_NC, _NS, _L = 2, 16, 16
_NW = _NC * _NS
_BPW = _BATCH // _NW
_PACK = 128 // _DIM

_LN2 = 0.6931471805599453
_SQRT2 = 1.4142135623730951


def _neg_log(p):
    bits = plsc.bitcast(p, jnp.int32)
    e = lax.shift_right_logical(bits, 23) - 127
    m = plsc.bitcast(
        (bits & jnp.int32(0x007FFFFF)) | jnp.int32(0x3F800000), jnp.float32
    )
    big = m > jnp.float32(_SQRT2)
    m = jnp.where(big, m * jnp.float32(0.5), m)
    e = jnp.where(big, e + 1, e)
    s = (m - jnp.float32(1.0)) / (m + jnp.float32(1.0))
    s2 = s * s
    lnm = s * (jnp.float32(2.0) + s2 * (jnp.float32(2.0 / 3.0)
               + s2 * (jnp.float32(2.0 / 5.0) + s2 * jnp.float32(2.0 / 7.0))))
    return -(e.astype(jnp.float32) * jnp.float32(_LN2) + lnm)


def _body(table_hbm, idxi_hbm, idxj_hbm, dummy_hbm, out_hbm,
          idxi_v, idxj_v, rows_i, rows_j, out_v, sem_i, sem_j):
    wid = lax.axis_index("s") * _NC + lax.axis_index("c")

    pltpu.sync_copy(idxi_hbm.at[wid], idxi_v)
    pltpu.sync_copy(idxj_hbm.at[wid], idxj_v)

    def fire(b, carry):
        vi16 = idxi_v[pl.ds(b * _L, _L)]
        vj16 = idxj_v[pl.ds(b * _L, _L)]
        for l in range(_L):
            r2 = b * (_L // _PACK) + l // _PACK
            c2 = (l % _PACK) * _DIM
            pltpu.async_copy(
                table_hbm.at[vi16[l]], rows_i.at[r2, pl.ds(c2, _DIM)], sem_i)
            pltpu.async_copy(
                table_hbm.at[vj16[l]], rows_j.at[r2, pl.ds(c2, _DIM)], sem_j)
        return carry

    lax.fori_loop(0, _BPW // _L, fire, None)

    lanes = lax.iota(jnp.int32, _L)
    par = (lanes & 1) * _DIM

    def group(g, carry):
        dummy8 = dummy_hbm.at[pl.ds(0, 8)]
        pltpu.make_async_copy(dummy8, rows_i.at[pl.ds(0, 8)], sem_i).wait()
        pltpu.make_async_copy(dummy8, rows_j.at[pl.ds(0, 8)], sem_j).wait()
        row2 = lax.shift_right_logical(lanes + g * _L, 1)
        acc = jnp.zeros((_L,), jnp.float32)
        for d in range(_DIM):
            col = par + d
            vi = plsc.load_gather(rows_i, [row2, col])
            vj = plsc.load_gather(rows_j, [row2, col])
            acc = acc + vi * vj
        p = jnp.float32(1.0) / (jnp.float32(1.0) + jnp.exp(-acc))
        p = jnp.maximum(p, jnp.float32(1e-8))
        out_v[pl.ds(g * _L, _L)] = _neg_log(p)
        return carry

    lax.fori_loop(0, _BPW // _L, group, None)
    pltpu.sync_copy(out_v, out_hbm.at[pl.ds(wid * _BPW, _BPW)])


@functools.partial(
    pl.kernel,
    out_type=jax.ShapeDtypeStruct((_BATCH,), jnp.float32),
    mesh=plsc.VectorSubcoreMesh(
        core_axis_name="c", subcore_axis_name="s",
        num_cores=_NC, num_subcores=_NS),
    scratch_types=[
        pltpu.VMEM((_BPW,), jnp.int32),
        pltpu.VMEM((_BPW,), jnp.int32),
        pltpu.VMEM((_BPW // _PACK, 128), jnp.float32),
        pltpu.VMEM((_BPW // _PACK, 128), jnp.float32),
        pltpu.VMEM((_BPW,), jnp.float32),
        pltpu.SemaphoreType.DMA,
        pltpu.SemaphoreType.DMA,
    ],
    compiler_params=pltpu.CompilerParams(needs_layout_passes=False),
)
def _sc_loss(table_hbm, idxi_hbm, idxj_hbm, dummy_hbm, out_hbm, *scratch):
    _body(table_hbm, idxi_hbm, idxj_hbm, dummy_hbm, out_hbm, *scratch)


def kernel(node_i, node_j, table):
    idx_i = node_i.astype(jnp.int32).reshape(_NW, _BPW)
    idx_j = node_j.astype(jnp.int32).reshape(_NW, _BPW)
    dummy = jnp.zeros((_BPW // _PACK, 128), jnp.float32)
    return _sc_loss(table, idx_i, idx_j, dummy)


# speedup vs baseline: 1.7740x; 1.0014x over previous
import functools

import jax
import jax.numpy as jnp
from jax import lax
from jax.experimental import pallas as pl
from jax.experimental.pallas import tpu as pltpu
from jax.experimental.pallas import tpu_sc as plsc

_NUM_NODES = 1000000
_DIM = 64
_BATCH = 16384
_NC, _NS, _L = 2, 16, 16
_NW = _NC * _NS
_BPW = _BATCH // _NW
_PACK = 128 // _DIM

_LN2 = 0.6931471805599453
_SQRT2 = 1.4142135623730951


def _neg_log(p):
    bits = plsc.bitcast(p, jnp.int32)
    e = lax.shift_right_logical(bits, 23) - 127
    m = plsc.bitcast(
        (bits & jnp.int32(0x007FFFFF)) | jnp.int32(0x3F800000), jnp.float32
    )
    big = m > jnp.float32(_SQRT2)
    m = jnp.where(big, m * jnp.float32(0.5), m)
    e = jnp.where(big, e + 1, e)
    s = (m - jnp.float32(1.0)) / (m + jnp.float32(1.0))
    s2 = s * s
    lnm = s * (jnp.float32(2.0) + s2 * (jnp.float32(2.0 / 3.0)
               + s2 * (jnp.float32(2.0 / 5.0) + s2 * jnp.float32(2.0 / 7.0))))
    return -(e.astype(jnp.float32) * jnp.float32(_LN2) + lnm)


def _body(table_hbm, idxi_hbm, idxj_hbm, dummy_hbm, out_hbm,
          idxi_v, idxj_v, rows_i, rows_j, out_v, sem_i, sem_j):
    wid = lax.axis_index("s") * _NC + lax.axis_index("c")

    pltpu.sync_copy(idxi_hbm.at[wid], idxi_v)
    pltpu.sync_copy(idxj_hbm.at[wid], idxj_v)

    def fire(b, carry):
        vi16 = idxi_v[pl.ds(b * _L, _L)]
        vj16 = idxj_v[pl.ds(b * _L, _L)]
        for l in range(_L):
            r2 = b * (_L // _PACK) + l // _PACK
            c2 = (l % _PACK) * _DIM
            pltpu.async_copy(
                table_hbm.at[vi16[l]], rows_i.at[r2, pl.ds(c2, _DIM)], sem_i)
            pltpu.async_copy(
                table_hbm.at[vj16[l]], rows_j.at[r2, pl.ds(c2, _DIM)], sem_j)
        return carry

    lax.fori_loop(0, _BPW // _L, fire, None)
    pltpu.make_async_copy(dummy_hbm, rows_i, sem_i).wait()
    pltpu.make_async_copy(dummy_hbm, rows_j, sem_j).wait()

    lanes = lax.iota(jnp.int32, _L)
    par = (lanes & 1) * _DIM

    def group(g, carry):
        row2 = lax.shift_right_logical(lanes + g * _L, 1)
        acc = jnp.zeros((_L,), jnp.float32)
        for d in range(_DIM):
            col = par + d
            vi = plsc.load_gather(rows_i, [row2, col])
            vj = plsc.load_gather(rows_j, [row2, col])
            acc = acc + vi * vj
        p = jnp.float32(1.0) / (jnp.float32(1.0) + jnp.exp(-acc))
        p = jnp.maximum(p, jnp.float32(1e-8))
        out_v[pl.ds(g * _L, _L)] = _neg_log(p)
        return carry

    lax.fori_loop(0, _BPW // _L, group, None)
    pltpu.sync_copy(out_v, out_hbm.at[pl.ds(wid * _BPW, _BPW)])


@functools.partial(
    pl.kernel,
    out_type=jax.ShapeDtypeStruct((_BATCH,), jnp.float32),
    mesh=plsc.VectorSubcoreMesh(
        core_axis_name="c", subcore_axis_name="s",
        num_cores=_NC, num_subcores=_NS),
    scratch_types=[
        pltpu.VMEM((_BPW,), jnp.int32),
        pltpu.VMEM((_BPW,), jnp.int32),
        pltpu.VMEM((_BPW // _PACK, 128), jnp.float32),
        pltpu.VMEM((_BPW // _PACK, 128), jnp.float32),
        pltpu.VMEM((_BPW,), jnp.float32),
        pltpu.SemaphoreType.DMA,
        pltpu.SemaphoreType.DMA,
    ],
    compiler_params=pltpu.CompilerParams(needs_layout_passes=False),
)
def _sc_loss(table_hbm, idxi_hbm, idxj_hbm, dummy_hbm, out_hbm, *scratch):
    _body(table_hbm, idxi_hbm, idxj_hbm, dummy_hbm, out_hbm, *scratch)


def kernel(node_i, node_j, table):
    idx_i = node_i.astype(jnp.int32).reshape(_NW, _BPW)
    idx_j = node_j.astype(jnp.int32).reshape(_NW, _BPW)
    dummy = jnp.zeros((_BPW // _PACK, 128), jnp.float32)
    return _sc_loss(table, idx_i, idx_j, dummy)
